# baseline (speedup 1.0000x reference)
import functools

import jax
import jax.numpy as jnp
from jax import lax
from jax.experimental import pallas as pl
from jax.experimental.pallas import tpu as pltpu
from jax.experimental.pallas import tpu_sc as plsc

NC = 2
NS = 16
LANES = 16

CH = 128
G = 8
GE = G * CH

_DNUMS = lax.GatherDimensionNumbers(
    offset_dims=(), collapsed_slice_dims=(0,), start_index_map=(0,))


def _sc_segment_sum(feat, srcp, dst2d, wp, n_nodes, d, groups_per_tile):
    mesh = plsc.VectorSubcoreMesh(
        core_axis_name="c", subcore_axis_name="s", num_cores=NC,
        num_subcores=NS)
    rows_per_tile = (n_nodes // NS) & ~7
    tail_rows = n_nodes - NS * rows_per_tile
    ng2 = groups_per_tile // 2
    assert groups_per_tile % 2 == 0

    @functools.partial(
        pl.kernel,
        out_type=jax.ShapeDtypeStruct((NC, n_nodes, d), jnp.float32),
        mesh=mesh,
        scratch_types=[
            pltpu.VMEM_SHARED((n_nodes, d), jnp.float32),
            pltpu.VMEM((2, CH, d), jnp.float32),
            pltpu.VMEM((2, GE), jnp.int32),
            pltpu.VMEM((2, GE), jnp.float32),
            pltpu.VMEM((2, G, CH), jnp.int32),
            pltpu.SemaphoreType.DMA,
            pltpu.SemaphoreType.DMA,
            pltpu.SemaphoreType.DMA,
            pltpu.SemaphoreType.DMA,
        ],
    )
    def k(feat_hbm, src_hbm, dst_hbm, w_hbm, out_hbm,
          acc, rows2, srcg2, wg2, dstg2, gsem, ssem0, ssem1, isem):
        c = lax.axis_index("c")
        s = lax.axis_index("s")
        wid = c * NS + s
        tb = wid * (groups_per_tile * GE)
        rb = wid * (groups_per_tile * G)

        def stage_descs(gi, slot):
            eb = tb + gi * GE
            return [
                pltpu.make_async_copy(
                    src_hbm.at[pl.ds(eb, GE)], srcg2.at[slot], isem),
                pltpu.make_async_copy(
                    w_hbm.at[pl.ds(eb, GE)], wg2.at[slot], isem),
                pltpu.make_async_copy(
                    dst_hbm.at[pl.ds(rb + gi * G, G)], dstg2.at[slot],
                    isem),
            ]

        def gather_desc(slot, j, p):
            return pltpu.make_async_copy(
                feat_hbm.at[srcg2.at[slot, pl.ds(j * CH, CH)]],
                rows2.at[p], gsem)

        def scatter_desc(slot, j, p):
            return pltpu.make_async_copy(
                rows2.at[p], acc.at[dstg2.at[slot, j]],
                ssem0 if p == 0 else ssem1)

        for dsc in stage_descs(0, 0):
            dsc.start()
        for dsc in stage_descs(1, 1):
            dsc.start()

        def zero_row(e, carry):
            for d16 in range(d // LANES):
                rows2[0, e, pl.ds(d16 * LANES, LANES)] = jnp.zeros(
                    (LANES,), jnp.float32)
            return carry
        lax.fori_loop(0, CH, zero_row, None)
        base_row = s * rows_per_tile
        for i in range(rows_per_tile // CH):
            pltpu.sync_copy(rows2.at[0],
                            acc.at[pl.ds(base_row + i * CH, CH)])
        rem = rows_per_tile % CH
        if rem:
            pltpu.sync_copy(
                rows2.at[0, pl.ds(0, rem)],
                acc.at[pl.ds(base_row + rows_per_tile - rem, rem)])

        @pl.when(s == NS - 1)
        def _():
            pltpu.sync_copy(
                rows2.at[0, pl.ds(0, tail_rows)],
                acc.at[pl.ds(NS * rows_per_tile, tail_rows)])

        plsc.subcore_barrier()

        for dsc in stage_descs(0, 0):
            dsc.wait()
        gather_desc(0, 0, 0).start()

        def mul_chunk(q, j, p):
            rp = rows2.at[p]
            wq = wg2.at[q]

            def mul_g(g, carry):
                w16 = wq[pl.ds(j * CH + g * LANES, LANES)]

                def mul_r4(r4, carry2):
                    for rr in range(4):
                        r = r4 * 4 + rr
                        wvec = lax.gather(
                            w16, jnp.full((LANES, 1), r, jnp.int32),
                            _DNUMS, (1,),
                            mode=lax.GatherScatterMode.PROMISE_IN_BOUNDS)
                        e = g * LANES + r
                        for d16 in range(d // LANES):
                            sl = pl.ds(d16 * LANES, LANES)
                            rp[e, sl] = rp[e, sl] * wvec
                    return carry2
                return lax.fori_loop(0, 4, mul_r4, carry)
            lax.fori_loop(0, CH // LANES, mul_g, None)

        def body(gg, carry):
            for gq in range(2):
                for j in range(G):
                    p = j % 2
                    gather_desc(gq, j, p).wait()
                    mul_chunk(gq, j, p)
                    scatter_desc(gq, j, p).start(add=True)
                    if j >= 1:
                        scatter_desc(gq, j - 1, 1 - p).wait()
                    elif gq == 1:
                        scatter_desc(0, G - 1, 1 - p).wait()
                    else:
                        @pl.when(gg > 0)
                        def _():
                            scatter_desc(1, G - 1, 1 - p).wait()
                    if j < G - 1:
                        gather_desc(gq, j + 1, 1 - p).start()
                    elif gq == 0:
                        for dsc in stage_descs(2 * gg + 1, 1):
                            dsc.wait()
                        gather_desc(1, 0, 1 - p).start()

                        @pl.when(gg < ng2 - 1)
                        def _():
                            for dsc in stage_descs(2 * gg + 2, 0):
                                dsc.start()
                    else:
                        @pl.when(gg < ng2 - 1)
                        def _():
                            for dsc in stage_descs(2 * gg + 2, 0):
                                dsc.wait()
                            gather_desc(0, 0, 0).start()
                            for dsc in stage_descs(2 * gg + 3, 1):
                                dsc.start()
            return carry
        lax.fori_loop(0, ng2, body, None)

        scatter_desc(1, G - 1, 1).wait()
        plsc.subcore_barrier()
        pltpu.sync_copy(acc.at[pl.ds(base_row, rows_per_tile)],
                        out_hbm.at[c, pl.ds(base_row, rows_per_tile)])

        @pl.when(s == NS - 1)
        def _():
            pltpu.sync_copy(
                acc.at[pl.ds(NS * rows_per_tile, tail_rows)],
                out_hbm.at[c, pl.ds(NS * rows_per_tile, tail_rows)])

    return k(feat, srcp, dst2d, wp)


def _tc_linear(partials, W, b2d, n_nodes, d):
    blk = 1000

    def body(p_ref, w_ref, b_ref, o_ref):
        x = p_ref[0] + p_ref[1]
        y = lax.dot_general(x, w_ref[...], (((1,), (1,)), ((), ())),
                            preferred_element_type=jnp.float32)
        o_ref[...] = y + b_ref[...]

    return pl.pallas_call(
        body,
        grid=(n_nodes // blk,),
        in_specs=[
            pl.BlockSpec((NC, blk, d), lambda i: (0, i, 0)),
            pl.BlockSpec((d, d), lambda i: (0, 0)),
            pl.BlockSpec((1, d), lambda i: (0, 0)),
        ],
        out_specs=pl.BlockSpec((blk, d), lambda i: (i, 0)),
        out_shape=jax.ShapeDtypeStruct((n_nodes, d), jnp.float32),
    )(partials, W, b2d)


def kernel(feat, edge_index, edge_weight, W, b):
    n_nodes, d = feat.shape
    n_edges = edge_index.shape[1]

    tile_quant = 2 * GE
    edges_per_tile = ((n_edges + NC * NS * tile_quant - 1)
                      // (NC * NS * tile_quant)) * tile_quant
    e_pad = NC * NS * edges_per_tile
    pad = e_pad - n_edges
    spread = jnp.arange(pad, dtype=jnp.int32) % n_nodes
    src = jnp.concatenate([edge_index[0], spread])
    dst = jnp.concatenate([edge_index[1], spread])
    w = jnp.concatenate(
        [edge_weight, jnp.zeros((pad,), jnp.float32)])
    dst2d = dst.reshape(e_pad // CH, CH)

    partials = _sc_segment_sum(feat, src, dst2d, w, n_nodes, d,
                               edges_per_tile // GE)
    return _tc_linear(partials, W, b.reshape(1, d), n_nodes, d)


# speedup vs baseline: 1.6998x; 1.6998x over previous
import functools

import jax
import jax.numpy as jnp
from jax import lax
from jax.experimental import pallas as pl
from jax.experimental.pallas import tpu as pltpu
from jax.experimental.pallas import tpu_sc as plsc

NC = 2
NS = 16
LANES = 16

CH = 64
G = 16
GE = G * CH

_DNUMS = lax.GatherDimensionNumbers(
    offset_dims=(), collapsed_slice_dims=(0,), start_index_map=(0,))


def _sc_segment_sum(feat, srcp, dst2d, wp, n_nodes, d, groups_per_tile):
    mesh = plsc.VectorSubcoreMesh(
        core_axis_name="c", subcore_axis_name="s", num_cores=NC,
        num_subcores=NS)
    rows_per_tile = (n_nodes // NS) & ~7
    tail_rows = n_nodes - NS * rows_per_tile
    ng2 = groups_per_tile // 2
    assert groups_per_tile % 2 == 0

    @functools.partial(
        pl.kernel,
        out_type=jax.ShapeDtypeStruct((NC, n_nodes, d), jnp.float32),
        mesh=mesh,
        scratch_types=[
            pltpu.VMEM_SHARED((n_nodes, d), jnp.float32),
            pltpu.VMEM((2, CH, d), jnp.float32),
            pltpu.VMEM((2, CH, d), jnp.float32),
            pltpu.VMEM((2, GE), jnp.int32),
            pltpu.VMEM((2, GE), jnp.float32),
            pltpu.VMEM((2, G, CH), jnp.int32),
            pltpu.SemaphoreType.DMA,
            pltpu.SemaphoreType.DMA,
            pltpu.SemaphoreType.DMA,
            pltpu.SemaphoreType.DMA,
        ],
    )
    def k(feat_hbm, src_hbm, dst_hbm, w_hbm, out_hbm,
          acc, rowsa, rowsb, srcg2, wg2, dstg2, gsem, ssem0, ssem1, isem):
        c = lax.axis_index("c")
        s = lax.axis_index("s")
        wid = c * NS + s
        tb = wid * (groups_per_tile * GE)
        rb = wid * (groups_per_tile * G)

        def stage_descs(gi, slot):
            eb = tb + gi * GE
            return [
                pltpu.make_async_copy(
                    src_hbm.at[pl.ds(eb, GE)], srcg2.at[slot], isem),
                pltpu.make_async_copy(
                    w_hbm.at[pl.ds(eb, GE)], wg2.at[slot], isem),
                pltpu.make_async_copy(
                    dst_hbm.at[pl.ds(rb + gi * G, G)], dstg2.at[slot],
                    isem),
            ]

        def gather_desc(slot, j, p):
            return pltpu.make_async_copy(
                feat_hbm.at[srcg2.at[slot, pl.ds(j * CH, CH)]],
                rowsa.at[p], gsem)

        def scatter_desc(slot, j, p):
            return pltpu.make_async_copy(
                rowsb.at[p], acc.at[dstg2.at[slot, j]],
                ssem0 if p == 0 else ssem1)

        for dsc in stage_descs(0, 0):
            dsc.start()
        for dsc in stage_descs(1, 1):
            dsc.start()

        def zero_row(e, carry):
            for d16 in range(d // LANES):
                rowsb[0, e, pl.ds(d16 * LANES, LANES)] = jnp.zeros(
                    (LANES,), jnp.float32)
            return carry
        lax.fori_loop(0, CH, zero_row, None)
        base_row = s * rows_per_tile
        for i in range(rows_per_tile // CH):
            pltpu.sync_copy(rowsb.at[0],
                            acc.at[pl.ds(base_row + i * CH, CH)])
        rem = rows_per_tile % CH
        if rem:
            pltpu.sync_copy(
                rowsb.at[0, pl.ds(0, rem)],
                acc.at[pl.ds(base_row + rows_per_tile - rem, rem)])

        @pl.when(s == NS - 1)
        def _():
            pltpu.sync_copy(
                rowsb.at[0, pl.ds(0, tail_rows)],
                acc.at[pl.ds(NS * rows_per_tile, tail_rows)])

        plsc.subcore_barrier()

        for dsc in stage_descs(0, 0):
            dsc.wait()
        gather_desc(0, 0, 0).start()

        dl = d // LANES
        LOOKAHEAD = 6

        def mul_chunk(q, j, p):
            ra = rowsa.at[p]
            rb_ = rowsb.at[p]
            wq = wg2.at[q]

            def bcast(w16, r):
                return lax.gather(
                    w16, jnp.full((LANES, 1), r, jnp.int32), _DNUMS, (1,),
                    mode=lax.GatherScatterMode.PROMISE_IN_BOUNDS)

            def mul_g(g, carry):
                w16 = wq[pl.ds(j * CH + g * LANES, LANES)]
                nsl = LANES * dl
                vals = [None] * nsl
                prods = [None] * nsl
                wv = [None] * LANES

                def addr(kk):
                    return g * LANES + kk // dl, pl.ds(
                        (kk % dl) * LANES, LANES)

                wv[0] = bcast(w16, 0)
                for kk in range(LOOKAHEAD):
                    e, sl = addr(kk)
                    vals[kk] = ra[e, sl]
                for kk in range(nsl):
                    if kk + LOOKAHEAD < nsl:
                        e, sl = addr(kk + LOOKAHEAD)
                        vals[kk + LOOKAHEAD] = ra[e, sl]
                    if (kk + 2) % dl == 0 and (kk + 2) // dl < LANES:
                        r = (kk + 2) // dl
                        wv[r] = bcast(w16, r)
                    prods[kk] = vals[kk] * wv[kk // dl]
                    if kk >= 1:
                        e, sl = addr(kk - 1)
                        rb_[e, sl] = prods[kk - 1]
                e, sl = addr(nsl - 1)
                rb_[e, sl] = prods[nsl - 1]
                return carry
            lax.fori_loop(0, CH // LANES, mul_g, None)

        def body(gg, carry):
            for gq in range(2):
                def pair(j2, carry2):
                    ja = 2 * j2
                    jb = ja + 1

                    gather_desc(gq, ja, 0).wait()
                    gather_desc(gq, jb, 1).start()
                    @pl.when(j2 >= 1)
                    def _():
                        scatter_desc(gq, ja - 2, 0).wait()
                    if gq == 1:
                        @pl.when(j2 == 0)
                        def _():
                            scatter_desc(0, G - 2, 0).wait()
                    else:
                        @pl.when(jnp.logical_and(j2 == 0, gg > 0))
                        def _():
                            scatter_desc(1, G - 2, 0).wait()
                    mul_chunk(gq, ja, 0)
                    scatter_desc(gq, ja, 0).start(add=True)

                    gather_desc(gq, jb, 1).wait()

                    @pl.when(j2 < G // 2 - 1)
                    def _():
                        gather_desc(gq, jb + 1, 0).start()
                    if gq == 0:
                        @pl.when(j2 == G // 2 - 1)
                        def _():
                            for dsc in stage_descs(2 * gg + 1, 1):
                                dsc.wait()
                            gather_desc(1, 0, 0).start()

                            @pl.when(gg < ng2 - 1)
                            def _():
                                for dsc in stage_descs(2 * gg + 2, 0):
                                    dsc.start()
                    else:
                        @pl.when(jnp.logical_and(j2 == G // 2 - 1,
                                                 gg < ng2 - 1))
                        def _():
                            for dsc in stage_descs(2 * gg + 2, 0):
                                dsc.wait()
                            gather_desc(0, 0, 0).start()
                            for dsc in stage_descs(2 * gg + 3, 1):
                                dsc.start()

                    @pl.when(j2 >= 1)
                    def _():
                        scatter_desc(gq, jb - 2, 1).wait()
                    if gq == 1:
                        @pl.when(j2 == 0)
                        def _():
                            scatter_desc(0, G - 1, 1).wait()
                    else:
                        @pl.when(jnp.logical_and(j2 == 0, gg > 0))
                        def _():
                            scatter_desc(1, G - 1, 1).wait()
                    mul_chunk(gq, jb, 1)
                    scatter_desc(gq, jb, 1).start(add=True)
                    return carry2
                lax.fori_loop(0, G // 2, pair, None)
            return carry
        lax.fori_loop(0, ng2, body, None)

        scatter_desc(1, G - 2, 0).wait()
        scatter_desc(1, G - 1, 1).wait()
        plsc.subcore_barrier()
        pltpu.sync_copy(acc.at[pl.ds(base_row, rows_per_tile)],
                        out_hbm.at[c, pl.ds(base_row, rows_per_tile)])

        @pl.when(s == NS - 1)
        def _():
            pltpu.sync_copy(
                acc.at[pl.ds(NS * rows_per_tile, tail_rows)],
                out_hbm.at[c, pl.ds(NS * rows_per_tile, tail_rows)])

    return k(feat, srcp, dst2d, wp)


def _tc_linear(partials, W, b2d, n_nodes, d):
    blk = 1000

    def body(p_ref, w_ref, b_ref, o_ref):
        x = p_ref[0] + p_ref[1]
        y = lax.dot_general(x, w_ref[...], (((1,), (1,)), ((), ())),
                            preferred_element_type=jnp.float32)
        o_ref[...] = y + b_ref[...]

    return pl.pallas_call(
        body,
        grid=(n_nodes // blk,),
        in_specs=[
            pl.BlockSpec((NC, blk, d), lambda i: (0, i, 0)),
            pl.BlockSpec((d, d), lambda i: (0, 0)),
            pl.BlockSpec((1, d), lambda i: (0, 0)),
        ],
        out_specs=pl.BlockSpec((blk, d), lambda i: (i, 0)),
        out_shape=jax.ShapeDtypeStruct((n_nodes, d), jnp.float32),
    )(partials, W, b2d)


def kernel(feat, edge_index, edge_weight, W, b):
    n_nodes, d = feat.shape
    n_edges = edge_index.shape[1]

    tile_quant = 2 * GE
    edges_per_tile = ((n_edges + NC * NS * tile_quant - 1)
                      // (NC * NS * tile_quant)) * tile_quant
    e_pad = NC * NS * edges_per_tile
    pad = e_pad - n_edges
    spread = jnp.arange(pad, dtype=jnp.int32) % n_nodes
    src = jnp.concatenate([edge_index[0], spread])
    dst = jnp.concatenate([edge_index[1], spread])
    w = jnp.concatenate(
        [edge_weight, jnp.zeros((pad,), jnp.float32)])
    dst2d = dst.reshape(e_pad // CH, CH)

    partials = _sc_segment_sum(feat, src, dst2d, w, n_nodes, d,
                               edges_per_tile // GE)
    return _tc_linear(partials, W, b.reshape(1, d), n_nodes, d)


# speedup vs baseline: 1.7386x; 1.0229x over previous
import functools

import jax
import jax.numpy as jnp
from jax import lax
from jax.experimental import pallas as pl
from jax.experimental.pallas import tpu as pltpu
from jax.experimental.pallas import tpu_sc as plsc

NC = 2
NS = 16
LANES = 16

CH = 128
G = 8
GE = G * CH

_DNUMS = lax.GatherDimensionNumbers(
    offset_dims=(), collapsed_slice_dims=(0,), start_index_map=(0,))


def _sc_segment_sum(feat, srcp, dst2d, wp, n_nodes, d, groups_per_tile):
    mesh = plsc.VectorSubcoreMesh(
        core_axis_name="c", subcore_axis_name="s", num_cores=NC,
        num_subcores=NS)
    rows_per_tile = (n_nodes // NS) & ~7
    tail_rows = n_nodes - NS * rows_per_tile
    ng2 = groups_per_tile // 2
    assert groups_per_tile % 2 == 0

    @functools.partial(
        pl.kernel,
        out_type=jax.ShapeDtypeStruct((NC, n_nodes, d), jnp.float32),
        mesh=mesh,
        scratch_types=[
            pltpu.VMEM_SHARED((n_nodes, d), jnp.float32),
            pltpu.VMEM((2, CH, d), jnp.float32),
            pltpu.VMEM((2, GE), jnp.int32),
            pltpu.VMEM((2, GE), jnp.float32),
            pltpu.VMEM((2, G, CH), jnp.int32),
            pltpu.SemaphoreType.DMA,
            pltpu.SemaphoreType.DMA,
            pltpu.SemaphoreType.DMA,
            pltpu.SemaphoreType.DMA,
        ],
    )
    def k(feat_hbm, src_hbm, dst_hbm, w_hbm, out_hbm,
          acc, rows2, srcg2, wg2, dstg2, gsem, ssem0, ssem1, isem):
        c = lax.axis_index("c")
        s = lax.axis_index("s")
        wid = c * NS + s
        tb = wid * (groups_per_tile * GE)
        rb = wid * (groups_per_tile * G)

        def stage_descs(gi, slot):
            eb = tb + gi * GE
            return [
                pltpu.make_async_copy(
                    src_hbm.at[pl.ds(eb, GE)], srcg2.at[slot], isem),
                pltpu.make_async_copy(
                    w_hbm.at[pl.ds(eb, GE)], wg2.at[slot], isem),
                pltpu.make_async_copy(
                    dst_hbm.at[pl.ds(rb + gi * G, G)], dstg2.at[slot],
                    isem),
            ]

        def gather_desc(slot, j, p):
            return pltpu.make_async_copy(
                feat_hbm.at[srcg2.at[slot, pl.ds(j * CH, CH)]],
                rows2.at[p], gsem)

        def scatter_desc(slot, j, p):
            return pltpu.make_async_copy(
                rows2.at[p], acc.at[dstg2.at[slot, j]],
                ssem0 if p == 0 else ssem1)

        for dsc in stage_descs(0, 0):
            dsc.start()
        for dsc in stage_descs(1, 1):
            dsc.start()

        def zero_row(e, carry):
            for d16 in range(d // LANES):
                rows2[0, e, pl.ds(d16 * LANES, LANES)] = jnp.zeros(
                    (LANES,), jnp.float32)
            return carry
        lax.fori_loop(0, CH, zero_row, None)
        base_row = s * rows_per_tile
        for i in range(rows_per_tile // CH):
            pltpu.sync_copy(rows2.at[0],
                            acc.at[pl.ds(base_row + i * CH, CH)])
        rem = rows_per_tile % CH
        if rem:
            pltpu.sync_copy(
                rows2.at[0, pl.ds(0, rem)],
                acc.at[pl.ds(base_row + rows_per_tile - rem, rem)])

        @pl.when(s == NS - 1)
        def _():
            pltpu.sync_copy(
                rows2.at[0, pl.ds(0, tail_rows)],
                acc.at[pl.ds(NS * rows_per_tile, tail_rows)])

        plsc.subcore_barrier()

        for dsc in stage_descs(0, 0):
            dsc.wait()
        gather_desc(0, 0, 0).start()

        dl = d // LANES
        LOOKAHEAD = 6

        def mul_chunk(q, j, p):
            rp = rows2.at[p]
            wq = wg2.at[q]

            def bcast(w16, r):
                return lax.gather(
                    w16, jnp.full((LANES, 1), r, jnp.int32), _DNUMS, (1,),
                    mode=lax.GatherScatterMode.PROMISE_IN_BOUNDS)

            def mul_g(g, carry):
                w16 = wq[pl.ds(j * CH + g * LANES, LANES)]
                nsl = LANES * dl
                vals = [None] * nsl
                prods = [None] * nsl
                wv = [None] * LANES

                def addr(kk):
                    return g * LANES + kk // dl, pl.ds(
                        (kk % dl) * LANES, LANES)

                wv[0] = bcast(w16, 0)
                for kk in range(LOOKAHEAD):
                    e, sl = addr(kk)
                    vals[kk] = rp[e, sl]
                for kk in range(nsl):
                    if kk + LOOKAHEAD < nsl:
                        e, sl = addr(kk + LOOKAHEAD)
                        vals[kk + LOOKAHEAD] = rp[e, sl]
                    if (kk + 2) % dl == 0 and (kk + 2) // dl < LANES:
                        r = (kk + 2) // dl
                        wv[r] = bcast(w16, r)
                    prods[kk] = vals[kk] * wv[kk // dl]
                    if kk >= 1:
                        e, sl = addr(kk - 1)
                        rp[e, sl] = prods[kk - 1]
                e, sl = addr(nsl - 1)
                rp[e, sl] = prods[nsl - 1]
                return carry
            lax.fori_loop(0, CH // LANES, mul_g, None)

        def body(gg, carry):
            for gq in range(2):
                def pair(j2, carry2):
                    ja = 2 * j2
                    jb = ja + 1

                    gather_desc(gq, ja, 0).wait()
                    mul_chunk(gq, ja, 0)
                    scatter_desc(gq, ja, 0).start(add=True)
                    @pl.when(j2 >= 1)
                    def _():
                        scatter_desc(gq, jb - 2, 1).wait()
                    if gq == 1:
                        @pl.when(j2 == 0)
                        def _():
                            scatter_desc(0, G - 1, 1).wait()
                    else:
                        @pl.when(jnp.logical_and(j2 == 0, gg > 0))
                        def _():
                            scatter_desc(1, G - 1, 1).wait()
                    gather_desc(gq, jb, 1).start()

                    gather_desc(gq, jb, 1).wait()
                    mul_chunk(gq, jb, 1)
                    scatter_desc(gq, jb, 1).start(add=True)
                    scatter_desc(gq, ja, 0).wait()

                    @pl.when(j2 < G // 2 - 1)
                    def _():
                        gather_desc(gq, jb + 1, 0).start()
                    if gq == 0:
                        @pl.when(j2 == G // 2 - 1)
                        def _():
                            for dsc in stage_descs(2 * gg + 1, 1):
                                dsc.wait()
                            gather_desc(1, 0, 0).start()

                            @pl.when(gg < ng2 - 1)
                            def _():
                                for dsc in stage_descs(2 * gg + 2, 0):
                                    dsc.start()
                    else:
                        @pl.when(jnp.logical_and(j2 == G // 2 - 1,
                                                 gg < ng2 - 1))
                        def _():
                            for dsc in stage_descs(2 * gg + 2, 0):
                                dsc.wait()
                            gather_desc(0, 0, 0).start()
                            for dsc in stage_descs(2 * gg + 3, 1):
                                dsc.start()
                    return carry2
                lax.fori_loop(0, G // 2, pair, None)
            return carry
        lax.fori_loop(0, ng2, body, None)

        scatter_desc(1, G - 1, 1).wait()
        plsc.subcore_barrier()
        pltpu.sync_copy(acc.at[pl.ds(base_row, rows_per_tile)],
                        out_hbm.at[c, pl.ds(base_row, rows_per_tile)])

        @pl.when(s == NS - 1)
        def _():
            pltpu.sync_copy(
                acc.at[pl.ds(NS * rows_per_tile, tail_rows)],
                out_hbm.at[c, pl.ds(NS * rows_per_tile, tail_rows)])

    return k(feat, srcp, dst2d, wp)


def _tc_linear(partials, W, b2d, n_nodes, d):
    blk = 1000

    def body(p_ref, w_ref, b_ref, o_ref):
        x = p_ref[0] + p_ref[1]
        y = lax.dot_general(x, w_ref[...], (((1,), (1,)), ((), ())),
                            preferred_element_type=jnp.float32)
        o_ref[...] = y + b_ref[...]

    return pl.pallas_call(
        body,
        grid=(n_nodes // blk,),
        in_specs=[
            pl.BlockSpec((NC, blk, d), lambda i: (0, i, 0)),
            pl.BlockSpec((d, d), lambda i: (0, 0)),
            pl.BlockSpec((1, d), lambda i: (0, 0)),
        ],
        out_specs=pl.BlockSpec((blk, d), lambda i: (i, 0)),
        out_shape=jax.ShapeDtypeStruct((n_nodes, d), jnp.float32),
    )(partials, W, b2d)


def kernel(feat, edge_index, edge_weight, W, b):
    n_nodes, d = feat.shape
    n_edges = edge_index.shape[1]

    tile_quant = 2 * GE
    edges_per_tile = ((n_edges + NC * NS * tile_quant - 1)
                      // (NC * NS * tile_quant)) * tile_quant
    e_pad = NC * NS * edges_per_tile
    pad = e_pad - n_edges
    spread = jnp.arange(pad, dtype=jnp.int32) % n_nodes
    src = jnp.concatenate([edge_index[0], spread])
    dst = jnp.concatenate([edge_index[1], spread])
    w = jnp.concatenate(
        [edge_weight, jnp.zeros((pad,), jnp.float32)])
    dst2d = dst.reshape(e_pad // CH, CH)

    partials = _sc_segment_sum(feat, src, dst2d, w, n_nodes, d,
                               edges_per_tile // GE)
    return _tc_linear(partials, W, b.reshape(1, d), n_nodes, d)


# speedup vs baseline: 1.8372x; 1.0567x over previous
import functools

import jax
import jax.numpy as jnp
from jax import lax
from jax.experimental import pallas as pl
from jax.experimental.pallas import tpu as pltpu
from jax.experimental.pallas import tpu_sc as plsc

NC = 2
NS = 16
LANES = 16

CH = 128
G = 8
GE = G * CH

_DNUMS = lax.GatherDimensionNumbers(
    offset_dims=(), collapsed_slice_dims=(0,), start_index_map=(0,))


def _sc_segment_sum(feat, srcp, dst2d, wp, n_nodes, d, groups_per_tile):
    mesh = plsc.VectorSubcoreMesh(
        core_axis_name="c", subcore_axis_name="s", num_cores=NC,
        num_subcores=NS)
    rows_per_tile = (n_nodes // NS) & ~7
    tail_rows = n_nodes - NS * rows_per_tile
    ng2 = groups_per_tile // 2
    assert groups_per_tile % 2 == 0

    @functools.partial(
        pl.kernel,
        out_type=jax.ShapeDtypeStruct((NC, n_nodes, d), jnp.float32),
        mesh=mesh,
        scratch_types=[
            pltpu.VMEM_SHARED((n_nodes, d), jnp.float32),
            pltpu.VMEM((2, CH, d), jnp.float32),
            pltpu.VMEM((2, GE), jnp.int32),
            pltpu.VMEM((2, GE), jnp.float32),
            pltpu.VMEM((2, G, CH), jnp.int32),
            pltpu.SemaphoreType.DMA,
            pltpu.SemaphoreType.DMA,
            pltpu.SemaphoreType.DMA,
            pltpu.SemaphoreType.DMA,
        ],
    )
    def k(feat_hbm, src_hbm, dst_hbm, w_hbm, out_hbm,
          acc, rows2, srcg2, wg2, dstg2, gsem, ssem0, ssem1, isem):
        c = lax.axis_index("c")
        s = lax.axis_index("s")
        wid = c * NS + s
        tb = wid * (groups_per_tile * GE)
        rb = wid * (groups_per_tile * G)

        def stage_descs(gi, slot):
            eb = tb + gi * GE
            return [
                pltpu.make_async_copy(
                    src_hbm.at[pl.ds(eb, GE)], srcg2.at[slot], isem),
                pltpu.make_async_copy(
                    w_hbm.at[pl.ds(eb, GE)], wg2.at[slot], isem),
                pltpu.make_async_copy(
                    dst_hbm.at[pl.ds(rb + gi * G, G)], dstg2.at[slot],
                    isem),
            ]

        def gather_desc(slot, j, p):
            return pltpu.make_async_copy(
                feat_hbm.at[srcg2.at[slot, pl.ds(j * CH, CH)]],
                rows2.at[p], gsem)

        def scatter_desc(slot, j, p):
            return pltpu.make_async_copy(
                rows2.at[p], acc.at[dstg2.at[slot, j]],
                ssem0 if p == 0 else ssem1)

        for dsc in stage_descs(0, 0):
            dsc.start()
        for dsc in stage_descs(1, 1):
            dsc.start()

        def zero_row(e, carry):
            for d16 in range(d // LANES):
                rows2[0, e, pl.ds(d16 * LANES, LANES)] = jnp.zeros(
                    (LANES,), jnp.float32)
            return carry
        lax.fori_loop(0, CH, zero_row, None)
        base_row = s * rows_per_tile
        for i in range(rows_per_tile // CH):
            pltpu.sync_copy(rows2.at[0],
                            acc.at[pl.ds(base_row + i * CH, CH)])
        rem = rows_per_tile % CH
        if rem:
            pltpu.sync_copy(
                rows2.at[0, pl.ds(0, rem)],
                acc.at[pl.ds(base_row + rows_per_tile - rem, rem)])

        @pl.when(s == NS - 1)
        def _():
            pltpu.sync_copy(
                rows2.at[0, pl.ds(0, tail_rows)],
                acc.at[pl.ds(NS * rows_per_tile, tail_rows)])

        plsc.subcore_barrier()

        for dsc in stage_descs(0, 0):
            dsc.wait()
        gather_desc(0, 0, 0).start()

        dl = d // LANES
        LOOKAHEAD = 6

        def mul_chunk(q, j, p):
            rp = rows2.at[p]
            wq = wg2.at[q]

            def bcast(w16, r):
                return lax.gather(
                    w16, jnp.full((LANES, 1), r, jnp.int32), _DNUMS, (1,),
                    mode=lax.GatherScatterMode.PROMISE_IN_BOUNDS)

            def mul_g(g, carry):
                w16 = wq[pl.ds(j * CH + g * LANES, LANES)]
                nsl = LANES * dl
                vals = [None] * nsl
                prods = [None] * nsl
                wv = [None] * LANES

                def addr(kk):
                    return g * LANES + kk // dl, pl.ds(
                        (kk % dl) * LANES, LANES)

                wv[0] = bcast(w16, 0)
                for kk in range(LOOKAHEAD):
                    e, sl = addr(kk)
                    vals[kk] = rp[e, sl]
                for kk in range(nsl):
                    if kk + LOOKAHEAD < nsl:
                        e, sl = addr(kk + LOOKAHEAD)
                        vals[kk + LOOKAHEAD] = rp[e, sl]
                    if (kk + 2) % dl == 0 and (kk + 2) // dl < LANES:
                        r = (kk + 2) // dl
                        wv[r] = bcast(w16, r)
                    prods[kk] = vals[kk] * wv[kk // dl]
                    if kk >= 1:
                        e, sl = addr(kk - 1)
                        rp[e, sl] = prods[kk - 1]
                e, sl = addr(nsl - 1)
                rp[e, sl] = prods[nsl - 1]
                return carry
            lax.fori_loop(0, CH // LANES, mul_g, None)

        def body(gg, carry):
            for gq in range(2):
                def pair(j2, carry2):
                    ja = 2 * j2
                    jb = ja + 1

                    gather_desc(gq, ja, 0).wait()
                    @pl.when(j2 >= 1)
                    def _():
                        scatter_desc(gq, jb - 2, 1).wait()
                    if gq == 1:
                        @pl.when(j2 == 0)
                        def _():
                            scatter_desc(0, G - 1, 1).wait()
                    else:
                        @pl.when(jnp.logical_and(j2 == 0, gg > 0))
                        def _():
                            scatter_desc(1, G - 1, 1).wait()
                    gather_desc(gq, jb, 1).start()
                    mul_chunk(gq, ja, 0)
                    gather_desc(gq, jb, 1).wait()
                    scatter_desc(gq, ja, 0).start(add=True)

                    mul_chunk(gq, jb, 1)
                    scatter_desc(gq, jb, 1).start(add=True)
                    scatter_desc(gq, ja, 0).wait()

                    @pl.when(j2 < G // 2 - 1)
                    def _():
                        gather_desc(gq, jb + 1, 0).start()
                    if gq == 0:
                        @pl.when(j2 == G // 2 - 1)
                        def _():
                            for dsc in stage_descs(2 * gg + 1, 1):
                                dsc.wait()
                            gather_desc(1, 0, 0).start()

                            @pl.when(gg < ng2 - 1)
                            def _():
                                for dsc in stage_descs(2 * gg + 2, 0):
                                    dsc.start()
                    else:
                        @pl.when(jnp.logical_and(j2 == G // 2 - 1,
                                                 gg < ng2 - 1))
                        def _():
                            for dsc in stage_descs(2 * gg + 2, 0):
                                dsc.wait()
                            gather_desc(0, 0, 0).start()
                            for dsc in stage_descs(2 * gg + 3, 1):
                                dsc.start()
                    return carry2
                lax.fori_loop(0, G // 2, pair, None)
            return carry
        lax.fori_loop(0, ng2, body, None)

        scatter_desc(1, G - 1, 1).wait()
        plsc.subcore_barrier()
        pltpu.sync_copy(acc.at[pl.ds(base_row, rows_per_tile)],
                        out_hbm.at[c, pl.ds(base_row, rows_per_tile)])

        @pl.when(s == NS - 1)
        def _():
            pltpu.sync_copy(
                acc.at[pl.ds(NS * rows_per_tile, tail_rows)],
                out_hbm.at[c, pl.ds(NS * rows_per_tile, tail_rows)])

    return k(feat, srcp, dst2d, wp)


def _tc_linear(partials, W, b2d, n_nodes, d):
    blk = 1000

    def body(p_ref, w_ref, b_ref, o_ref):
        x = p_ref[0] + p_ref[1]
        y = lax.dot_general(x, w_ref[...], (((1,), (1,)), ((), ())),
                            preferred_element_type=jnp.float32)
        o_ref[...] = y + b_ref[...]

    return pl.pallas_call(
        body,
        grid=(n_nodes // blk,),
        in_specs=[
            pl.BlockSpec((NC, blk, d), lambda i: (0, i, 0)),
            pl.BlockSpec((d, d), lambda i: (0, 0)),
            pl.BlockSpec((1, d), lambda i: (0, 0)),
        ],
        out_specs=pl.BlockSpec((blk, d), lambda i: (i, 0)),
        out_shape=jax.ShapeDtypeStruct((n_nodes, d), jnp.float32),
    )(partials, W, b2d)


def kernel(feat, edge_index, edge_weight, W, b):
    n_nodes, d = feat.shape
    n_edges = edge_index.shape[1]

    tile_quant = 2 * GE
    edges_per_tile = ((n_edges + NC * NS * tile_quant - 1)
                      // (NC * NS * tile_quant)) * tile_quant
    e_pad = NC * NS * edges_per_tile
    pad = e_pad - n_edges
    spread = jnp.arange(pad, dtype=jnp.int32) % n_nodes
    src = jnp.concatenate([edge_index[0], spread])
    dst = jnp.concatenate([edge_index[1], spread])
    w = jnp.concatenate(
        [edge_weight, jnp.zeros((pad,), jnp.float32)])
    dst2d = dst.reshape(e_pad // CH, CH)

    partials = _sc_segment_sum(feat, src, dst2d, w, n_nodes, d,
                               edges_per_tile // GE)
    return _tc_linear(partials, W, b.reshape(1, d), n_nodes, d)
